# Initial kernel scaffold; baseline (speedup 1.0000x reference)
import functools

import jax
import jax.numpy as jnp
from jax import lax
from jax.experimental import pallas as pl
from jax.experimental.pallas import tpu as pltpu
from jax.experimental.pallas import tpu_sc as plsc

N = 10000
E = 320000
DIM = 128
NC = 2
NS = 16
NW = NC * NS
EP = E // NW
C = 80
NCHUNK = EP // C
RPT = N // NS
RPC = 125

_mesh = plsc.VectorSubcoreMesh(core_axis_name="c", subcore_axis_name="s")


@functools.partial(
    pl.kernel,
    out_type=jax.ShapeDtypeStruct((NC, N, DIM), jnp.float32),
    mesh=_mesh,
    scratch_types=[
        pltpu.VMEM_SHARED((N, DIM), jnp.float32),
        pltpu.VMEM((C,), jnp.int32),
        pltpu.VMEM((C,), jnp.int32),
        pltpu.VMEM((C,), jnp.float32),
        pltpu.VMEM((C, DIM), jnp.float32),
        pltpu.VMEM((RPC, DIM), jnp.float32),
        pltpu.SemaphoreType.DMA,
    ],
)
def _sc_scatter(cols_hbm, rows_hbm, vals_hbm, w_hbm, out_hbm,
                acc, colv, rowv, valv, gbuf, zbuf, sem):
    cid = lax.axis_index("c")
    sid = lax.axis_index("s")
    wid = cid * NS + sid

    for r in range(RPC):
        for j in range(DIM // 16):
            zbuf[r, pl.ds(j * 16, 16)] = jnp.zeros((16,), jnp.float32)
    for t in range(RPT // RPC):
        pltpu.sync_copy(zbuf, acc.at[pl.ds(sid * RPT + t * RPC, RPC)])
    plsc.subcore_barrier()

    base = wid * EP

    @pl.loop(0, NCHUNK)
    def _chunk(k):
        off = base + k * C
        pltpu.sync_copy(cols_hbm.at[pl.ds(off, C)], colv)
        pltpu.sync_copy(rows_hbm.at[pl.ds(off, C)], rowv)
        pltpu.sync_copy(vals_hbm.at[pl.ds(off, C)], valv)
        pltpu.async_copy(w_hbm.at[colv], gbuf, sem).wait()

        @pl.loop(0, C)
        def _edge(i):
            v = valv[i]
            for j in range(DIM // 16):
                sl = pl.ds(j * 16, 16)
                gbuf[i, sl] = gbuf[i, sl] * v

        pltpu.sync_copy(gbuf, acc.at[rowv], add=True)

    plsc.subcore_barrier()

    for t in range(RPT // RPC):
        r0 = sid * RPT + t * RPC
        pltpu.sync_copy(acc.at[pl.ds(r0, RPC)], zbuf)
        pltpu.sync_copy(zbuf, out_hbm.at[cid, pl.ds(r0, RPC)])


def _combine_body(p_ref, o_ref):
    o_ref[...] = jnp.maximum(p_ref[0] + p_ref[1], 0.0)


_combine = pl.pallas_call(
    _combine_body,
    grid=(10,),
    in_specs=[pl.BlockSpec((NC, N // 10, DIM), lambda i: (0, i, 0))],
    out_specs=pl.BlockSpec((N // 10, DIM), lambda i: (i, 0)),
    out_shape=jax.ShapeDtypeStruct((N, DIM), jnp.float32),
)


def kernel(edge_index, edge_values, w, inputs):
    del inputs
    rows = edge_index[0]
    cols = edge_index[1]
    partials = _sc_scatter(cols, rows, edge_values, w)
    return _combine(partials)


# speedup vs baseline: 4.5235x; 4.5235x over previous
import functools

import jax
import jax.numpy as jnp
from jax import lax
from jax.experimental import pallas as pl
from jax.experimental.pallas import tpu as pltpu
from jax.experimental.pallas import tpu_sc as plsc

N = 10000
E = 320000
DIM = 128
NC = 2
NS = 16
NW = NC * NS
EP = E // NW
C = 80
NCHUNK = EP // C
RPC = 80
NRC = N // RPC

_mesh = plsc.VectorSubcoreMesh(core_axis_name="c", subcore_axis_name="s")


@functools.partial(
    pl.kernel,
    out_type=jax.ShapeDtypeStruct((NC, N, DIM), jnp.float32),
    mesh=_mesh,
    scratch_types=[
        pltpu.VMEM_SHARED((N, DIM), jnp.float32),
        pltpu.VMEM((C,), jnp.int32),
        pltpu.VMEM((C,), jnp.int32),
        pltpu.VMEM((C,), jnp.float32),
        pltpu.VMEM((C, DIM), jnp.float32),
        pltpu.VMEM((RPC, DIM), jnp.float32),
        pltpu.SemaphoreType.DMA,
    ],
)
def _sc_scatter(cols_hbm, rows_hbm, vals_hbm, w_hbm, out_hbm,
                acc, colv, rowv, valv, gbuf, zbuf, sem):
    cid = lax.axis_index("c")
    sid = lax.axis_index("s")
    wid = cid * NS + sid

    for r in range(RPC):
        for j in range(DIM // 16):
            zbuf[r, pl.ds(j * 16, 16)] = jnp.zeros((16,), jnp.float32)
    nt = (NRC - sid + NS - 1) // NS

    @pl.loop(0, nt)
    def _zero(t):
        q = sid + t * NS
        pltpu.sync_copy(zbuf, acc.at[pl.ds(q * RPC, RPC)])

    plsc.subcore_barrier()

    base = wid * EP

    @pl.loop(0, NCHUNK)
    def _chunk(k):
        off = base + k * C
        pltpu.sync_copy(cols_hbm.at[pl.ds(off, C)], colv)
        pltpu.sync_copy(rows_hbm.at[pl.ds(off, C)], rowv)
        pltpu.sync_copy(vals_hbm.at[pl.ds(off, C)], valv)
        pltpu.async_copy(w_hbm.at[colv], gbuf, sem).wait()

        @pl.loop(0, C // 16)
        def _edge_group(g):
            val16 = valv[pl.ds(g * 16, 16)]
            for l in range(16):
                i = g * 16 + l
                v = val16[l]
                for j in range(DIM // 16):
                    sl = pl.ds(j * 16, 16)
                    gbuf[i, sl] = gbuf[i, sl] * v

        pltpu.sync_copy(gbuf, acc.at[rowv], add=True)

    plsc.subcore_barrier()

    @pl.loop(0, nt)
    def _out(t):
        r0 = (sid + t * NS) * RPC
        pltpu.sync_copy(acc.at[pl.ds(r0, RPC)], zbuf)
        pltpu.sync_copy(zbuf, out_hbm.at[cid, pl.ds(r0, RPC)])


def _combine_body(p_ref, o_ref):
    o_ref[...] = jnp.maximum(p_ref[0] + p_ref[1], 0.0)


_combine = pl.pallas_call(
    _combine_body,
    grid=(10,),
    in_specs=[pl.BlockSpec((NC, N // 10, DIM), lambda i: (0, i, 0))],
    out_specs=pl.BlockSpec((N // 10, DIM), lambda i: (i, 0)),
    out_shape=jax.ShapeDtypeStruct((N, DIM), jnp.float32),
)


def kernel(edge_index, edge_values, w, inputs):
    del inputs
    rows = edge_index[0]
    cols = edge_index[1]
    partials = _sc_scatter(cols, rows, edge_values, w)
    return _combine(partials)


# speedup vs baseline: 10.7326x; 2.3726x over previous
import functools

import jax
import jax.numpy as jnp
from jax import lax
from jax.experimental import pallas as pl
from jax.experimental.pallas import tpu as pltpu
from jax.experimental.pallas import tpu_sc as plsc

N = 10000
E = 320000
DIM = 128
NC = 2
NS = 16
NW = NC * NS
EP = E // NW
C = 80
NCHUNK = EP // C
NSUP = 5
SCK = NCHUNK // NSUP
SPAIR = (SCK - 1) // 2
RPC = 80
NRC = N // RPC

_mesh = plsc.VectorSubcoreMesh(core_axis_name="c", subcore_axis_name="s")


@functools.partial(
    pl.kernel,
    out_type=jax.ShapeDtypeStruct((NC, N, DIM), jnp.float32),
    mesh=_mesh,
    scratch_types=[
        pltpu.VMEM_SHARED((N, DIM), jnp.float32),
        pltpu.VMEM((SCK * C,), jnp.int32),
        pltpu.VMEM((SCK, C), jnp.int32),
        pltpu.VMEM((SCK * C,), jnp.float32),
        pltpu.VMEM((C, DIM), jnp.float32),
        pltpu.VMEM((C, DIM), jnp.float32),
        pltpu.VMEM((C, DIM), jnp.float32),
        pltpu.VMEM((C, DIM), jnp.float32),
        pltpu.VMEM((C * 16,), jnp.float32),
        pltpu.SemaphoreType.DMA,
        pltpu.SemaphoreType.DMA,
        pltpu.SemaphoreType.DMA,
        pltpu.SemaphoreType.DMA,
    ],
)
def _sc_scatter(cols_hbm, rows_hbm, vals_hbm, w_hbm, out_hbm,
                acc, colv, rowv, valv, gbuf0, gbuf1, sbuf0, sbuf1, vrep,
                gsem0, gsem1, ssem0, ssem1):
    cid = lax.axis_index("c")
    sid = lax.axis_index("s")
    wid = cid * NS + sid
    gbuf = (gbuf0, gbuf1)
    sbuf = (sbuf0, sbuf1)
    gsem = (gsem0, gsem1)
    ssem = (ssem0, ssem1)

    for r in range(RPC):
        for j in range(DIM // 16):
            sbuf0[r, pl.ds(j * 16, 16)] = jnp.zeros((16,), jnp.float32)
    nt = (NRC - sid + NS - 1) // NS

    @pl.loop(0, nt)
    def _zero(t):
        q = sid + t * NS
        pltpu.sync_copy(sbuf0, acc.at[pl.ds(q * RPC, RPC)])

    plsc.subcore_barrier()

    def start_gather(k, b):
        pltpu.async_copy(w_hbm.at[colv.at[pl.ds(k * C, C)]], gbuf[b], gsem[b])

    def wait_gather(k, b):
        pltpu.make_async_copy(w_hbm.at[colv.at[pl.ds(k * C, C)]], gbuf[b], gsem[b]).wait()

    def start_scatter(k, b):
        pltpu.async_copy(sbuf[b], acc.at[rowv.at[k]], ssem[b], add=True)

    def wait_scatter(k, b):
        pltpu.make_async_copy(sbuf[b], acc.at[rowv.at[k]], ssem[b]).wait()

    def scale(k, b):
        @pl.loop(0, C // 16)
        def _group(g):
            val16 = valv[pl.ds(k * C + g * 16, 16)]
            for l in range(16):
                vrep[pl.ds((g * 16 + l) * 16, 16)] = jnp.full((16,), val16[l], jnp.float32)

        @plsc.parallel_loop(0, C, unroll=8)
        def _edge(i):
            m = vrep[pl.ds(i * 16, 16)]
            for j in range(DIM // 16):
                sl = pl.ds(j * 16, 16)
                sbuf[b][i, sl] = gbuf[b][i, sl] * m

    @pl.loop(0, NSUP)
    def _sup(s):
        pltpu.sync_copy(cols_hbm.at[wid, s], colv)
        pltpu.sync_copy(rows_hbm.at[wid, s], rowv)
        pltpu.sync_copy(vals_hbm.at[wid, s], valv)

        start_gather(0, 0)
        start_gather(1, 1)

        @pl.loop(0, SPAIR)
        def _pair(kk):
            for b in range(2):
                k = kk * 2 + b
                wait_gather(k, b)

                @pl.when(kk > 0)
                def _drain():
                    wait_scatter(k, b)

                scale(k, b)

                @pl.when(k + 2 < SCK)
                def _next():
                    start_gather(k + 2, b)

                start_scatter(k, b)

        k_last = SCK - 1
        wait_gather(k_last, 0)
        wait_scatter(k_last, 0)
        scale(k_last, 0)
        start_scatter(k_last, 0)
        wait_scatter(k_last, 0)
        wait_scatter(k_last - 1, 1)

    plsc.subcore_barrier()

    @pl.loop(0, nt)
    def _out(t):
        r0 = (sid + t * NS) * RPC
        pltpu.sync_copy(acc.at[pl.ds(r0, RPC)], sbuf0)
        pltpu.sync_copy(sbuf0, out_hbm.at[cid, pl.ds(r0, RPC)])


def _combine_body(p_ref, o_ref):
    o_ref[...] = jnp.maximum(p_ref[0] + p_ref[1], 0.0)


_combine = pl.pallas_call(
    _combine_body,
    grid=(10,),
    in_specs=[pl.BlockSpec((NC, N // 10, DIM), lambda i: (0, i, 0))],
    out_specs=pl.BlockSpec((N // 10, DIM), lambda i: (i, 0)),
    out_shape=jax.ShapeDtypeStruct((N, DIM), jnp.float32),
)


def kernel(edge_index, edge_values, w, inputs):
    del inputs
    rows = edge_index[0].reshape(NW, NSUP, SCK, C)
    cols = edge_index[1].reshape(NW, NSUP, SCK * C)
    vals = edge_values.reshape(NW, NSUP, SCK * C)
    partials = _sc_scatter(cols, rows, vals, w)
    return _combine(partials)


# speedup vs baseline: 10.8961x; 1.0152x over previous
import functools

import jax
import jax.numpy as jnp
from jax import lax
from jax.experimental import pallas as pl
from jax.experimental.pallas import tpu as pltpu
from jax.experimental.pallas import tpu_sc as plsc

N = 10000
E = 320000
DIM = 128
NC = 2
NS = 16
NW = NC * NS
EP = E // NW
C = 80
NCHUNK = EP // C
NSUP = 5
SCK = NCHUNK // NSUP
SPAIR = (SCK - 1) // 2
RPC = 80
NRC = N // RPC

_mesh = plsc.VectorSubcoreMesh(core_axis_name="c", subcore_axis_name="s")


@functools.partial(
    pl.kernel,
    out_type=jax.ShapeDtypeStruct((NC, N, DIM), jnp.float32),
    mesh=_mesh,
    scratch_types=[
        pltpu.VMEM_SHARED((N, DIM), jnp.float32),
        pltpu.VMEM((SCK * C,), jnp.int32),
        pltpu.VMEM((SCK, C), jnp.int32),
        pltpu.VMEM((SCK * C,), jnp.float32),
        pltpu.VMEM((C, DIM), jnp.float32),
        pltpu.VMEM((C, DIM), jnp.float32),
        pltpu.VMEM((C, DIM), jnp.float32),
        pltpu.VMEM((C, DIM), jnp.float32),
        pltpu.VMEM((C * 16,), jnp.float32),
        pltpu.SemaphoreType.DMA,
        pltpu.SemaphoreType.DMA,
        pltpu.SemaphoreType.DMA,
        pltpu.SemaphoreType.DMA,
    ],
)
def _sc_scatter(cols_hbm, rows_hbm, vals_hbm, w_hbm, out_hbm,
                acc, colv, rowv, valv, gbuf0, gbuf1, sbuf0, sbuf1, vrep,
                gsem0, gsem1, ssem0, ssem1):
    cid = lax.axis_index("c")
    sid = lax.axis_index("s")
    wid = cid * NS + sid
    gbuf = (gbuf0, gbuf1)
    sbuf = (sbuf0, sbuf1)
    gsem = (gsem0, gsem1)
    ssem = (ssem0, ssem1)

    for r in range(RPC):
        for j in range(DIM // 16):
            sbuf0[r, pl.ds(j * 16, 16)] = jnp.zeros((16,), jnp.float32)
    nt = (NRC - sid + NS - 1) // NS

    @pl.loop(0, nt)
    def _zero(t):
        q = sid + t * NS
        pltpu.sync_copy(sbuf0, acc.at[pl.ds(q * RPC, RPC)])

    plsc.subcore_barrier()

    def start_gather(k, b):
        pltpu.async_copy(w_hbm.at[colv.at[pl.ds(k * C, C)]], gbuf[b], gsem[b])

    def wait_gather(k, b):
        pltpu.make_async_copy(w_hbm.at[colv.at[pl.ds(k * C, C)]], gbuf[b], gsem[b]).wait()

    def start_scatter(k, b):
        pltpu.async_copy(sbuf[b], acc.at[rowv.at[k]], ssem[b], add=True)

    def wait_scatter(k, b):
        pltpu.make_async_copy(sbuf[b], acc.at[rowv.at[k]], ssem[b]).wait()

    def scale(k, b):
        @pl.loop(0, C // 16)
        def _group(g):
            val16 = valv[pl.ds(k * C + g * 16, 16)]
            for l in range(16):
                vrep[pl.ds((g * 16 + l) * 16, 16)] = jnp.full((16,), val16[l], jnp.float32)

        @plsc.parallel_loop(0, C, unroll=2)
        def _edge(i):
            m = vrep[pl.ds(i * 16, 16)]
            for j in range(DIM // 16):
                sl = pl.ds(j * 16, 16)
                sbuf[b][i, sl] = gbuf[b][i, sl] * m

    @pl.loop(0, NSUP)
    def _sup(s):
        pltpu.sync_copy(cols_hbm.at[wid, s], colv)
        pltpu.sync_copy(rows_hbm.at[wid, s], rowv)
        pltpu.sync_copy(vals_hbm.at[wid, s], valv)

        start_gather(0, 0)
        start_gather(1, 1)

        @pl.loop(0, SPAIR)
        def _pair(kk):
            for b in range(2):
                k = kk * 2 + b
                wait_gather(k, b)

                @pl.when(kk > 0)
                def _drain():
                    wait_scatter(k, b)

                scale(k, b)

                @pl.when(k + 2 < SCK)
                def _next():
                    start_gather(k + 2, b)

                start_scatter(k, b)

        k_last = SCK - 1
        wait_gather(k_last, 0)
        wait_scatter(k_last, 0)
        scale(k_last, 0)
        start_scatter(k_last, 0)
        wait_scatter(k_last, 0)
        wait_scatter(k_last - 1, 1)

    plsc.subcore_barrier()

    @pl.loop(0, nt)
    def _out(t):
        r0 = (sid + t * NS) * RPC
        pltpu.sync_copy(acc.at[pl.ds(r0, RPC)], sbuf0)
        pltpu.sync_copy(sbuf0, out_hbm.at[cid, pl.ds(r0, RPC)])


def _combine_body(p_ref, o_ref):
    o_ref[...] = jnp.maximum(p_ref[0] + p_ref[1], 0.0)


_combine = pl.pallas_call(
    _combine_body,
    grid=(10,),
    in_specs=[pl.BlockSpec((NC, N // 10, DIM), lambda i: (0, i, 0))],
    out_specs=pl.BlockSpec((N // 10, DIM), lambda i: (i, 0)),
    out_shape=jax.ShapeDtypeStruct((N, DIM), jnp.float32),
)


def kernel(edge_index, edge_values, w, inputs):
    del inputs
    rows = edge_index[0].reshape(NW, NSUP, SCK, C)
    cols = edge_index[1].reshape(NW, NSUP, SCK * C)
    vals = edge_values.reshape(NW, NSUP, SCK * C)
    partials = _sc_scatter(cols, rows, vals, w)
    return _combine(partials)


# speedup vs baseline: 10.9055x; 1.0009x over previous
import functools

import jax
import jax.numpy as jnp
from jax import lax
from jax.experimental import pallas as pl
from jax.experimental.pallas import tpu as pltpu
from jax.experimental.pallas import tpu_sc as plsc

N = 10000
E = 320000
DIM = 128
NC = 2
NS = 16
NW = NC * NS
EP = E // NW
C = 80
NCHUNK = EP // C
NSUP = 5
SCK = NCHUNK // NSUP
SPAIR = (SCK - 1) // 2
RPC = 80
NRC = N // RPC

_mesh = plsc.VectorSubcoreMesh(core_axis_name="c", subcore_axis_name="s")


@functools.partial(
    pl.kernel,
    out_type=jax.ShapeDtypeStruct((NC, N, DIM), jnp.float32),
    mesh=_mesh,
    scratch_types=[
        pltpu.VMEM_SHARED((N, DIM), jnp.float32),
        pltpu.VMEM((SCK * C,), jnp.int32),
        pltpu.VMEM((SCK, C), jnp.int32),
        pltpu.VMEM((SCK * C,), jnp.float32),
        pltpu.VMEM((C, DIM), jnp.float32),
        pltpu.VMEM((C, DIM), jnp.float32),
        pltpu.VMEM((C, DIM), jnp.float32),
        pltpu.VMEM((C, DIM), jnp.float32),
        pltpu.VMEM((C * 16,), jnp.float32),
        pltpu.SemaphoreType.DMA,
        pltpu.SemaphoreType.DMA,
        pltpu.SemaphoreType.DMA,
        pltpu.SemaphoreType.DMA,
    ],
)
def _sc_scatter(cols_hbm, rows_hbm, vals_hbm, w_hbm, out_hbm,
                acc, colv, rowv, valv, gbuf0, gbuf1, sbuf0, sbuf1, vrep,
                gsem0, gsem1, ssem0, ssem1):
    cid = lax.axis_index("c")
    sid = lax.axis_index("s")
    wid = cid * NS + sid
    gbuf = (gbuf0, gbuf1)
    sbuf = (sbuf0, sbuf1)
    gsem = (gsem0, gsem1)
    ssem = (ssem0, ssem1)

    for r in range(RPC):
        for j in range(DIM // 16):
            sbuf0[r, pl.ds(j * 16, 16)] = jnp.zeros((16,), jnp.float32)
    nt = (NRC - sid + NS - 1) // NS

    @pl.loop(0, nt)
    def _zero(t):
        q = sid + t * NS
        pltpu.sync_copy(sbuf0, acc.at[pl.ds(q * RPC, RPC)])

    plsc.subcore_barrier()

    def start_gather(k, b):
        pltpu.async_copy(w_hbm.at[colv.at[pl.ds(k * C, C)]], gbuf[b], gsem[b])

    def wait_gather(k, b):
        pltpu.make_async_copy(w_hbm.at[colv.at[pl.ds(k * C, C)]], gbuf[b], gsem[b]).wait()

    def start_scatter(k, b):
        pltpu.async_copy(sbuf[b], acc.at[rowv.at[k]], ssem[b], add=True)

    def wait_scatter(k, b):
        pltpu.make_async_copy(sbuf[b], acc.at[rowv.at[k]], ssem[b]).wait()

    def scale(k, b):
        @pl.loop(0, C // 16)
        def _group(g):
            val16 = valv[pl.ds(k * C + g * 16, 16)]
            for l in range(16):
                vrep[pl.ds((g * 16 + l) * 16, 16)] = jnp.full((16,), val16[l], jnp.float32)

        @plsc.parallel_loop(0, C, unroll=4)
        def _edge(i):
            m = vrep[pl.ds(i * 16, 16)]
            for j in range(DIM // 16):
                sl = pl.ds(j * 16, 16)
                sbuf[b][i, sl] = gbuf[b][i, sl] * m

    @pl.loop(0, NSUP)
    def _sup(s):
        pltpu.sync_copy(cols_hbm.at[wid, s], colv)
        pltpu.sync_copy(rows_hbm.at[wid, s], rowv)
        pltpu.sync_copy(vals_hbm.at[wid, s], valv)

        start_gather(0, 0)
        start_gather(1, 1)

        @pl.loop(0, SPAIR)
        def _pair(kk):
            for b in range(2):
                k = kk * 2 + b
                wait_gather(k, b)

                @pl.when(kk > 0)
                def _drain():
                    wait_scatter(k, b)

                scale(k, b)

                @pl.when(k + 2 < SCK)
                def _next():
                    start_gather(k + 2, b)

                start_scatter(k, b)

        k_last = SCK - 1
        wait_gather(k_last, 0)
        wait_scatter(k_last, 0)
        scale(k_last, 0)
        start_scatter(k_last, 0)
        wait_scatter(k_last, 0)
        wait_scatter(k_last - 1, 1)

    plsc.subcore_barrier()

    @pl.loop(0, nt)
    def _out(t):
        r0 = (sid + t * NS) * RPC
        pltpu.sync_copy(acc.at[pl.ds(r0, RPC)], sbuf0)
        pltpu.sync_copy(sbuf0, out_hbm.at[cid, pl.ds(r0, RPC)])


def _combine_body(p_ref, o_ref):
    o_ref[...] = jnp.maximum(p_ref[0] + p_ref[1], 0.0)


_combine = pl.pallas_call(
    _combine_body,
    grid=(10,),
    in_specs=[pl.BlockSpec((NC, N // 10, DIM), lambda i: (0, i, 0))],
    out_specs=pl.BlockSpec((N // 10, DIM), lambda i: (i, 0)),
    out_shape=jax.ShapeDtypeStruct((N, DIM), jnp.float32),
)


def kernel(edge_index, edge_values, w, inputs):
    del inputs
    rows = edge_index[0].reshape(NW, NSUP, SCK, C)
    cols = edge_index[1].reshape(NW, NSUP, SCK * C)
    vals = edge_values.reshape(NW, NSUP, SCK * C)
    partials = _sc_scatter(cols, rows, vals, w)
    return _combine(partials)
